# baseline (speedup 1.0000x reference)
import functools

import jax
import jax.numpy as jnp
from jax import lax
from jax.experimental import pallas as pl
from jax.experimental.pallas import tpu as pltpu
from jax.experimental.pallas import tpu_sc as plsc

B, S, D, FF, E, K = 2, 2048, 1024, 2048, 8, 2
N = B * S
TM = 256
NPAD = N * K + E * TM
TILES = NPAD // TM
TE_LANES = 48

NC, NS = 2, 16
NW = NC * NS
TPW = N // NW
CH = 16
NCH = TPW // CH

_SQRT_HALF = 0.7071067811865476


def _router_body(x_ref, gw_ref, gb_ref, i1_ref, i2_ref, w1o_ref, w2o_ref,
                 starts_ref, te_ref, nt_ref, cnt_ref):
    i = pl.program_id(0)
    nb = pl.num_programs(0)

    @pl.when(i == 0)
    def _():
        cnt_ref[...] = jnp.zeros_like(cnt_ref)

    x = x_ref[...]
    gw = gw_ref[...]
    logits = jax.lax.dot_general(
        x, gw, (((1,), (1,)), ((), ())),
        preferred_element_type=jnp.float32)
    logits = logits + gb_ref[...]
    iota = lax.broadcasted_iota(jnp.int32, logits.shape, 1)
    v1 = jnp.max(logits, axis=1, keepdims=True)
    i1 = jnp.min(jnp.where(logits == v1, iota, E), axis=1, keepdims=True)
    masked = jnp.where(iota == i1, -jnp.inf, logits)
    v2 = jnp.max(masked, axis=1, keepdims=True)
    i2 = jnp.min(jnp.where(masked == v2, iota, E), axis=1, keepdims=True)
    e1 = 1.0 / (1.0 + jnp.exp(v2 - v1))

    i1_ref[...] = i1
    i2_ref[...] = i2
    w1o_ref[...] = e1
    w2o_ref[...] = 1.0 - e1

    oh = (iota == i1).astype(jnp.float32) + (iota == i2).astype(jnp.float32)
    cnt_ref[...] += jnp.sum(oh, axis=0, keepdims=True)

    @pl.when(i == nb - 1)
    def _():
        cnt = cnt_ref[...]
        pc = jnp.ceil(cnt * (1.0 / TM)) * TM
        r8 = lax.broadcasted_iota(jnp.int32, (E, E), 0)
        c8 = lax.broadcasted_iota(jnp.int32, (E, E), 1)
        upper = (r8 < c8).astype(jnp.float32)
        starts = jax.lax.dot_general(
            pc, upper, (((1,), (0,)), ((), ())),
            preferred_element_type=jnp.float32)
        starts_ref[...] = starts.astype(jnp.int32)
        eye = (r8 == c8).astype(jnp.float32)
        st_col = jax.lax.dot_general(
            eye, starts * (1.0 / TM), (((1,), (1,)), ((), ())),
            preferred_element_type=jnp.float32)
        lane = lax.broadcasted_iota(jnp.int32, (E, TE_LANES), 1)
        ge = (lane >= st_col.astype(jnp.int32)).astype(jnp.int32)
        te_ref[...] = jnp.sum(ge, axis=0, keepdims=True) - 1
        nt_ref[...] = (jnp.sum(pc, axis=1, keepdims=True)
                       * (1.0 / TM)).astype(jnp.int32)


def _pos_body(i1_ref, i2_ref, starts_ref, p0_ref, p1_ref, carry_ref):
    b = pl.program_id(0)

    @pl.when(b == 0)
    def _():
        carry_ref[...] = jnp.zeros_like(carry_ref)

    nrows = i1_ref.shape[0]
    iota = lax.broadcasted_iota(jnp.int32, (nrows, E), 1)
    oh1 = (i1_ref[...] == iota).astype(jnp.float32)
    oh2 = (i2_ref[...] == iota).astype(jnp.float32)
    oht = oh1 + oh2
    r_io = lax.broadcasted_iota(jnp.int32, (nrows, nrows), 0)
    c_io = lax.broadcasted_iota(jnp.int32, (nrows, nrows), 1)
    ltri = (r_io >= c_io).astype(jnp.float32)
    incl = jax.lax.dot_general(ltri, oht, (((1,), (0,)), ((), ())),
                               preferred_element_type=jnp.float32)
    excl = incl - oht
    base = starts_ref[...].astype(jnp.float32) + carry_ref[...]
    p0 = jnp.sum(oh1 * (base + excl), axis=1, keepdims=True)
    p1 = jnp.sum(oh2 * (base + excl + oh1), axis=1, keepdims=True)
    p0_ref[...] = p0.astype(jnp.int32)
    p1_ref[...] = p1.astype(jnp.int32)
    carry_ref[...] += jnp.sum(oht, axis=0, keepdims=True)


def _dispatch_body(x_hbm, p0_hbm, p1_hbm, ww0_hbm, ww1_hbm, xs_hbm, ws_hbm,
                   idx0_v, idx1_v, rows_v, wv_v, wbuf_v, sem):
    wid = lax.axis_index("s") * NC + lax.axis_index("c")
    pltpu.sync_copy(p0_hbm.at[wid], idx0_v)
    pltpu.sync_copy(p1_hbm.at[wid], idx1_v)
    lanes = lax.iota(jnp.int32, 16)
    zeros = jnp.zeros((16,), jnp.int32)
    for c in range(NCH):
        tok0 = wid * TPW + c * CH
        pltpu.sync_copy(x_hbm.at[pl.ds(tok0, CH)], rows_v)
        pltpu.async_copy(rows_v, xs_hbm.at[idx0_v.at[c]], sem).wait()
        pltpu.async_copy(rows_v, xs_hbm.at[idx1_v.at[c]], sem).wait()
        pltpu.sync_copy(ww0_hbm.at[pl.ds(tok0, CH)], wv_v)
        plsc.store_scatter(wbuf_v, [lanes, zeros], wv_v[...])
        pltpu.async_copy(wbuf_v, ws_hbm.at[idx0_v.at[c]], sem).wait()
        pltpu.sync_copy(ww1_hbm.at[pl.ds(tok0, CH)], wv_v)
        plsc.store_scatter(wbuf_v, [lanes, zeros], wv_v[...])
        pltpu.async_copy(wbuf_v, ws_hbm.at[idx1_v.at[c]], sem).wait()


def _ffn_body(te_ref, nt_ref, xs_ref, ws_ref, w1_ref, b1_ref, w2_ref,
              b2_ref, w3_ref, b3_ref, out_ref):
    i = pl.program_id(0)

    @pl.when(i < nt_ref[0])
    def _():
        xb = xs_ref[...]
        w1 = w1_ref[0]
        w2 = w2_ref[0]
        w3 = w3_ref[0]
        h1 = jax.lax.dot_general(xb, w1, (((1,), (1,)), ((), ())),
                                 preferred_element_type=jnp.float32)
        h1 = h1 + b1_ref[0]
        h2 = jax.lax.dot_general(xb, w2, (((1,), (1,)), ((), ())),
                                 preferred_element_type=jnp.float32)
        h2 = h2 + b2_ref[0]
        g = 0.5 * h1 * (1.0 + jax.lax.erf(h1 * _SQRT_HALF))
        h = g * h2
        part = jax.lax.dot_general(h, w3, (((1,), (1,)), ((), ())),
                                   preferred_element_type=jnp.float32)
        out_ref[...] = ws_ref[:, 0:1] * (part + b3_ref[0])


def _combine_body(ys_hbm, p0_hbm, p1_hbm, g0_hbm, g1_hbm,
                  idx0_v, idx1_v, rows_v, sem):
    wid = lax.axis_index("s") * NC + lax.axis_index("c")
    pltpu.sync_copy(p0_hbm.at[wid], idx0_v)
    pltpu.sync_copy(p1_hbm.at[wid], idx1_v)
    for c in range(NCH):
        tok0 = wid * TPW + c * CH
        pltpu.async_copy(ys_hbm.at[idx0_v.at[c]], rows_v, sem).wait()
        pltpu.sync_copy(rows_v, g0_hbm.at[pl.ds(tok0, CH)])
        pltpu.async_copy(ys_hbm.at[idx1_v.at[c]], rows_v, sem).wait()
        pltpu.sync_copy(rows_v, g1_hbm.at[pl.ds(tok0, CH)])


def _add_body(a_ref, b_ref, o_ref):
    o_ref[...] = a_ref[...] + b_ref[...]


def kernel(x, gate_W, gate_b, w1_W, w1_b, w2_W, w2_b, w3_W, w3_b):
    xf = x.reshape(N, D)

    TN = 1024
    i1, i2, ww1, ww2, starts, te, nt = pl.pallas_call(
        _router_body,
        grid=(N // TN,),
        in_specs=[
            pl.BlockSpec((TN, D), lambda i: (i, 0)),
            pl.BlockSpec((E, D), lambda i: (0, 0)),
            pl.BlockSpec((1, E), lambda i: (0, 0)),
        ],
        out_specs=[
            pl.BlockSpec((TN, 1), lambda i: (i, 0)),
            pl.BlockSpec((TN, 1), lambda i: (i, 0)),
            pl.BlockSpec((TN, 1), lambda i: (i, 0)),
            pl.BlockSpec((TN, 1), lambda i: (i, 0)),
            pl.BlockSpec((1, E), lambda i: (0, 0)),
            pl.BlockSpec((1, TE_LANES), lambda i: (0, 0)),
            pl.BlockSpec((1, 1), lambda i: (0, 0)),
        ],
        out_shape=[
            jax.ShapeDtypeStruct((N, 1), jnp.int32),
            jax.ShapeDtypeStruct((N, 1), jnp.int32),
            jax.ShapeDtypeStruct((N, 1), jnp.float32),
            jax.ShapeDtypeStruct((N, 1), jnp.float32),
            jax.ShapeDtypeStruct((1, E), jnp.int32),
            jax.ShapeDtypeStruct((1, TE_LANES), jnp.int32),
            jax.ShapeDtypeStruct((1, 1), jnp.int32),
        ],
        scratch_shapes=[pltpu.VMEM((1, E), jnp.float32)],
    )(xf, gate_W, gate_b.reshape(1, E))

    TB = 512
    p0, p1 = pl.pallas_call(
        _pos_body,
        grid=(N // TB,),
        in_specs=[
            pl.BlockSpec((TB, 1), lambda b: (b, 0)),
            pl.BlockSpec((TB, 1), lambda b: (b, 0)),
            pl.BlockSpec((1, E), lambda b: (0, 0)),
        ],
        out_specs=[
            pl.BlockSpec((TB, 1), lambda b: (b, 0)),
            pl.BlockSpec((TB, 1), lambda b: (b, 0)),
        ],
        out_shape=[
            jax.ShapeDtypeStruct((N, 1), jnp.int32),
            jax.ShapeDtypeStruct((N, 1), jnp.int32),
        ],
        scratch_shapes=[pltpu.VMEM((1, E), jnp.float32)],
    )(i1, i2, starts)

    p0w = p0.reshape(NW, NCH, CH)
    p1w = p1.reshape(NW, NCH, CH)

    mesh = plsc.VectorSubcoreMesh(core_axis_name="c", subcore_axis_name="s")
    dispatch = functools.partial(
        pl.kernel,
        out_type=(jax.ShapeDtypeStruct((NPAD, D), jnp.float32),
                  jax.ShapeDtypeStruct((NPAD, 128), jnp.float32)),
        mesh=mesh,
        scratch_types=[
            pltpu.VMEM((NCH, CH), jnp.int32),
            pltpu.VMEM((NCH, CH), jnp.int32),
            pltpu.VMEM((CH, D), jnp.float32),
            pltpu.VMEM((CH,), jnp.float32),
            pltpu.VMEM((CH, 128), jnp.float32),
            pltpu.SemaphoreType.DMA,
        ],
        compiler_params=pltpu.CompilerParams(needs_layout_passes=False),
    )(_dispatch_body)
    xs, ws = dispatch(xf, p0w, p1w, ww1.reshape(N), ww2.reshape(N))

    ys = pl.pallas_call(
        _ffn_body,
        grid_spec=pltpu.PrefetchScalarGridSpec(
            num_scalar_prefetch=2,
            grid=(TILES,),
            in_specs=[
                pl.BlockSpec((TM, D), lambda i, te, nt: (i, 0)),
                pl.BlockSpec((TM, 128), lambda i, te, nt: (i, 0)),
                pl.BlockSpec((1, FF, D), lambda i, te, nt: (te[i], 0, 0)),
                pl.BlockSpec((1, 1, FF), lambda i, te, nt: (te[i], 0, 0)),
                pl.BlockSpec((1, FF, D), lambda i, te, nt: (te[i], 0, 0)),
                pl.BlockSpec((1, 1, FF), lambda i, te, nt: (te[i], 0, 0)),
                pl.BlockSpec((1, D, FF), lambda i, te, nt: (te[i], 0, 0)),
                pl.BlockSpec((1, 1, D), lambda i, te, nt: (te[i], 0, 0)),
            ],
            out_specs=pl.BlockSpec((TM, D), lambda i, te, nt: (i, 0)),
        ),
        out_shape=jax.ShapeDtypeStruct((NPAD, D), jnp.float32),
        compiler_params=pltpu.CompilerParams(
            vmem_limit_bytes=100 * 1024 * 1024),
    )(te.reshape(TE_LANES), nt.reshape(1), xs, ws,
      w1_W, w1_b.reshape(E, 1, FF), w2_W, w2_b.reshape(E, 1, FF),
      w3_W, w3_b.reshape(E, 1, D))

    combine = functools.partial(
        pl.kernel,
        out_type=(jax.ShapeDtypeStruct((N, D), jnp.float32),
                  jax.ShapeDtypeStruct((N, D), jnp.float32)),
        mesh=mesh,
        scratch_types=[
            pltpu.VMEM((NCH, CH), jnp.int32),
            pltpu.VMEM((NCH, CH), jnp.int32),
            pltpu.VMEM((CH, D), jnp.float32),
            pltpu.SemaphoreType.DMA,
        ],
        compiler_params=pltpu.CompilerParams(needs_layout_passes=False),
    )(_combine_body)
    g0, g1 = combine(ys, p0w, p1w)

    TA = 512
    out = pl.pallas_call(
        _add_body,
        grid=(N // TA,),
        in_specs=[
            pl.BlockSpec((TA, D), lambda i: (i, 0)),
            pl.BlockSpec((TA, D), lambda i: (i, 0)),
        ],
        out_specs=pl.BlockSpec((TA, D), lambda i: (i, 0)),
        out_shape=jax.ShapeDtypeStruct((N, D), jnp.float32),
    )(g0, g1)

    final = out.reshape(B, S, D)
    aux_loss = jnp.asarray(0.0, dtype=jnp.float32)
    return (final, aux_loss)


# speedup vs baseline: 1.0845x; 1.0845x over previous
import functools

import jax
import jax.numpy as jnp
from jax import lax
from jax.experimental import pallas as pl
from jax.experimental.pallas import tpu as pltpu
from jax.experimental.pallas import tpu_sc as plsc

B, S, D, FF, E, K = 2, 2048, 1024, 2048, 8, 2
N = B * S
TM = 256
NPAD = N * K + E * TM
TILES = NPAD // TM
TE_LANES = 48

NC, NS = 2, 16
NW = NC * NS
TPW = N // NW
CH = 16
NCH = TPW // CH
CH2 = 32
NCH2 = TPW // CH2

_SQRT_HALF = 0.7071067811865476


def _router_body(x_ref, gw_ref, gb_ref, i1_ref, i2_ref, w1o_ref, w2o_ref,
                 starts_ref, te_ref, nt_ref, cnt_ref):
    i = pl.program_id(0)
    nb = pl.num_programs(0)

    @pl.when(i == 0)
    def _():
        cnt_ref[...] = jnp.zeros_like(cnt_ref)

    x = x_ref[...]
    gw = gw_ref[...]
    logits = jax.lax.dot_general(
        x, gw, (((1,), (1,)), ((), ())),
        preferred_element_type=jnp.float32)
    logits = logits + gb_ref[...]
    iota = lax.broadcasted_iota(jnp.int32, logits.shape, 1)
    v1 = jnp.max(logits, axis=1, keepdims=True)
    i1 = jnp.min(jnp.where(logits == v1, iota, E), axis=1, keepdims=True)
    masked = jnp.where(iota == i1, -jnp.inf, logits)
    v2 = jnp.max(masked, axis=1, keepdims=True)
    i2 = jnp.min(jnp.where(masked == v2, iota, E), axis=1, keepdims=True)
    e1 = 1.0 / (1.0 + jnp.exp(v2 - v1))

    i1_ref[...] = i1
    i2_ref[...] = i2
    w1o_ref[...] = e1
    w2o_ref[...] = 1.0 - e1

    oh = (iota == i1).astype(jnp.float32) + (iota == i2).astype(jnp.float32)
    cnt_ref[...] += jnp.sum(oh, axis=0, keepdims=True)

    @pl.when(i == nb - 1)
    def _():
        cnt = cnt_ref[...]
        pc = jnp.ceil(cnt * (1.0 / TM)) * TM
        r8 = lax.broadcasted_iota(jnp.int32, (E, E), 0)
        c8 = lax.broadcasted_iota(jnp.int32, (E, E), 1)
        upper = (r8 < c8).astype(jnp.float32)
        starts = jax.lax.dot_general(
            pc, upper, (((1,), (0,)), ((), ())),
            preferred_element_type=jnp.float32)
        starts_ref[...] = starts.astype(jnp.int32)
        eye = (r8 == c8).astype(jnp.float32)
        st_col = jax.lax.dot_general(
            eye, starts * (1.0 / TM), (((1,), (1,)), ((), ())),
            preferred_element_type=jnp.float32)
        lane = lax.broadcasted_iota(jnp.int32, (E, TE_LANES), 1)
        ge = (lane >= st_col.astype(jnp.int32)).astype(jnp.int32)
        te_ref[...] = jnp.sum(ge, axis=0, keepdims=True) - 1
        nt_ref[...] = (jnp.sum(pc, axis=1, keepdims=True)
                       * (1.0 / TM)).astype(jnp.int32)


def _pos_body(i1_ref, i2_ref, starts_ref, p0_ref, p1_ref, carry_ref):
    b = pl.program_id(0)

    @pl.when(b == 0)
    def _():
        carry_ref[...] = jnp.zeros_like(carry_ref)

    nrows = i1_ref.shape[0]
    iota = lax.broadcasted_iota(jnp.int32, (nrows, E), 1)
    oh1 = (i1_ref[...] == iota).astype(jnp.float32)
    oh2 = (i2_ref[...] == iota).astype(jnp.float32)
    oht = oh1 + oh2
    r_io = lax.broadcasted_iota(jnp.int32, (nrows, nrows), 0)
    c_io = lax.broadcasted_iota(jnp.int32, (nrows, nrows), 1)
    ltri = (r_io >= c_io).astype(jnp.float32)
    incl = jax.lax.dot_general(ltri, oht, (((1,), (0,)), ((), ())),
                               preferred_element_type=jnp.float32)
    excl = incl - oht
    base = starts_ref[...].astype(jnp.float32) + carry_ref[...]
    p0 = jnp.sum(oh1 * (base + excl), axis=1, keepdims=True)
    p1 = jnp.sum(oh2 * (base + excl + oh1), axis=1, keepdims=True)
    p0_ref[...] = p0.astype(jnp.int32)
    p1_ref[...] = p1.astype(jnp.int32)
    carry_ref[...] += jnp.sum(oht, axis=0, keepdims=True)


def _dispatch_body(x_hbm, p0_hbm, p1_hbm, ww0_hbm, ww1_hbm, xs_hbm, ws_hbm,
                   idx0_v, idx1_v, rows0_v, rows1_v, wv0_v, wv1_v,
                   wbuf0_v, wbuf1_v, load_sem, scat_sem):
    wid = lax.axis_index("s") * NC + lax.axis_index("c")
    pltpu.sync_copy(p0_hbm.at[wid], idx0_v)
    pltpu.sync_copy(p1_hbm.at[wid], idx1_v)
    pltpu.sync_copy(ww0_hbm.at[pl.ds(wid * TPW, TPW)], wv0_v)
    pltpu.sync_copy(ww1_hbm.at[pl.ds(wid * TPW, TPW)], wv1_v)
    rows = [rows0_v, rows1_v]
    wbufs = [wbuf0_v, wbuf1_v]
    lanes = lax.iota(jnp.int32, 16)
    zeros = jnp.zeros((16,), jnp.int32)
    loads = [None] * NCH2
    scats = [None] * NCH2
    loads[0] = pltpu.async_copy(
        x_hbm.at[pl.ds(wid * TPW, CH2)], rows[0], load_sem)
    for c in range(NCH2):
        if c >= 1:
            for h in scats[c - 1]:
                h.wait()
        if c + 1 < NCH2:
            tok1 = wid * TPW + (c + 1) * CH2
            loads[c + 1] = pltpu.async_copy(
                x_hbm.at[pl.ds(tok1, CH2)], rows[(c + 1) % 2], load_sem)
        loads[c].wait()
        wb = wbufs[c % 2]
        for h in range(CH2 // 16):
            off = c * CH2 + h * 16
            plsc.store_scatter(wb, [lanes + h * 16, zeros],
                               wv0_v[pl.ds(off, 16)])
            plsc.store_scatter(wb, [lanes + CH2 + h * 16, zeros],
                               wv1_v[pl.ds(off, 16)])
        scats[c] = [
            pltpu.async_copy(rows[c % 2], xs_hbm.at[idx0_v.at[c]], scat_sem),
            pltpu.async_copy(rows[c % 2], xs_hbm.at[idx1_v.at[c]], scat_sem),
            pltpu.async_copy(wb.at[pl.ds(0, CH2)], ws_hbm.at[idx0_v.at[c]],
                             scat_sem),
            pltpu.async_copy(wb.at[pl.ds(CH2, CH2)], ws_hbm.at[idx1_v.at[c]],
                             scat_sem),
        ]
    for h in scats[NCH2 - 1]:
        h.wait()


def _ffn_body(te_ref, nt_ref, xs_ref, ws_ref, w1_ref, b1_ref, w2_ref,
              b2_ref, w3_ref, b3_ref, out_ref):
    i = pl.program_id(0)

    @pl.when(i < nt_ref[0])
    def _():
        xb = xs_ref[...]
        w1 = w1_ref[0]
        w2 = w2_ref[0]
        w3 = w3_ref[0]
        h1 = jax.lax.dot_general(xb, w1, (((1,), (1,)), ((), ())),
                                 preferred_element_type=jnp.float32)
        h1 = h1 + b1_ref[0]
        h2 = jax.lax.dot_general(xb, w2, (((1,), (1,)), ((), ())),
                                 preferred_element_type=jnp.float32)
        h2 = h2 + b2_ref[0]
        g = 0.5 * h1 * (1.0 + jax.lax.erf(h1 * _SQRT_HALF))
        h = g * h2
        part = jax.lax.dot_general(h, w3, (((1,), (1,)), ((), ())),
                                   preferred_element_type=jnp.float32)
        out_ref[...] = ws_ref[:, 0:1] * (part + b3_ref[0])


def _combine_body(ys_hbm, p0_hbm, p1_hbm, g0_hbm, g1_hbm,
                  idx0_v, idx1_v, r0a_v, r0b_v, r1a_v, r1b_v,
                  gat_sem, st_sem):
    wid = lax.axis_index("s") * NC + lax.axis_index("c")
    pltpu.sync_copy(p0_hbm.at[wid], idx0_v)
    pltpu.sync_copy(p1_hbm.at[wid], idx1_v)
    r0 = [r0a_v, r0b_v]
    r1 = [r1a_v, r1b_v]
    gats = [None] * NCH
    stores = [None] * NCH
    gats[0] = [
        pltpu.async_copy(ys_hbm.at[idx0_v.at[0]], r0[0], gat_sem),
        pltpu.async_copy(ys_hbm.at[idx1_v.at[0]], r1[0], gat_sem),
    ]
    for c in range(NCH):
        if c >= 1:
            for h in stores[c - 1]:
                h.wait()
        if c + 1 < NCH:
            gats[c + 1] = [
                pltpu.async_copy(ys_hbm.at[idx0_v.at[c + 1]],
                                 r0[(c + 1) % 2], gat_sem),
                pltpu.async_copy(ys_hbm.at[idx1_v.at[c + 1]],
                                 r1[(c + 1) % 2], gat_sem),
            ]
        for h in gats[c]:
            h.wait()
        tok0 = wid * TPW + c * CH
        stores[c] = [
            pltpu.async_copy(r0[c % 2], g0_hbm.at[pl.ds(tok0, CH)], st_sem),
            pltpu.async_copy(r1[c % 2], g1_hbm.at[pl.ds(tok0, CH)], st_sem),
        ]
    for h in stores[NCH - 1]:
        h.wait()


def _add_body(a_ref, b_ref, o_ref):
    o_ref[...] = a_ref[...] + b_ref[...]


def kernel(x, gate_W, gate_b, w1_W, w1_b, w2_W, w2_b, w3_W, w3_b):
    xf = x.reshape(N, D)

    TN = 1024
    i1, i2, ww1, ww2, starts, te, nt = pl.pallas_call(
        _router_body,
        grid=(N // TN,),
        in_specs=[
            pl.BlockSpec((TN, D), lambda i: (i, 0)),
            pl.BlockSpec((E, D), lambda i: (0, 0)),
            pl.BlockSpec((1, E), lambda i: (0, 0)),
        ],
        out_specs=[
            pl.BlockSpec((TN, 1), lambda i: (i, 0)),
            pl.BlockSpec((TN, 1), lambda i: (i, 0)),
            pl.BlockSpec((TN, 1), lambda i: (i, 0)),
            pl.BlockSpec((TN, 1), lambda i: (i, 0)),
            pl.BlockSpec((1, E), lambda i: (0, 0)),
            pl.BlockSpec((1, TE_LANES), lambda i: (0, 0)),
            pl.BlockSpec((1, 1), lambda i: (0, 0)),
        ],
        out_shape=[
            jax.ShapeDtypeStruct((N, 1), jnp.int32),
            jax.ShapeDtypeStruct((N, 1), jnp.int32),
            jax.ShapeDtypeStruct((N, 1), jnp.float32),
            jax.ShapeDtypeStruct((N, 1), jnp.float32),
            jax.ShapeDtypeStruct((1, E), jnp.int32),
            jax.ShapeDtypeStruct((1, TE_LANES), jnp.int32),
            jax.ShapeDtypeStruct((1, 1), jnp.int32),
        ],
        scratch_shapes=[pltpu.VMEM((1, E), jnp.float32)],
    )(xf, gate_W, gate_b.reshape(1, E))

    TB = 512
    p0, p1 = pl.pallas_call(
        _pos_body,
        grid=(N // TB,),
        in_specs=[
            pl.BlockSpec((TB, 1), lambda b: (b, 0)),
            pl.BlockSpec((TB, 1), lambda b: (b, 0)),
            pl.BlockSpec((1, E), lambda b: (0, 0)),
        ],
        out_specs=[
            pl.BlockSpec((TB, 1), lambda b: (b, 0)),
            pl.BlockSpec((TB, 1), lambda b: (b, 0)),
        ],
        out_shape=[
            jax.ShapeDtypeStruct((N, 1), jnp.int32),
            jax.ShapeDtypeStruct((N, 1), jnp.int32),
        ],
        scratch_shapes=[pltpu.VMEM((1, E), jnp.float32)],
    )(i1, i2, starts)

    p0w = p0.reshape(NW, NCH, CH)
    p1w = p1.reshape(NW, NCH, CH)
    p0d = p0.reshape(NW, NCH2, CH2)
    p1d = p1.reshape(NW, NCH2, CH2)

    mesh = plsc.VectorSubcoreMesh(core_axis_name="c", subcore_axis_name="s")
    dispatch = functools.partial(
        pl.kernel,
        out_type=(jax.ShapeDtypeStruct((NPAD, D), jnp.float32),
                  jax.ShapeDtypeStruct((NPAD, 128), jnp.float32)),
        mesh=mesh,
        scratch_types=[
            pltpu.VMEM((NCH2, CH2), jnp.int32),
            pltpu.VMEM((NCH2, CH2), jnp.int32),
            pltpu.VMEM((CH2, D), jnp.float32),
            pltpu.VMEM((CH2, D), jnp.float32),
            pltpu.VMEM((TPW,), jnp.float32),
            pltpu.VMEM((TPW,), jnp.float32),
            pltpu.VMEM((2 * CH2, 128), jnp.float32),
            pltpu.VMEM((2 * CH2, 128), jnp.float32),
            pltpu.SemaphoreType.DMA,
            pltpu.SemaphoreType.DMA,
        ],
        compiler_params=pltpu.CompilerParams(needs_layout_passes=False),
    )(_dispatch_body)
    xs, ws = dispatch(xf, p0d, p1d, ww1.reshape(N), ww2.reshape(N))

    ys = pl.pallas_call(
        _ffn_body,
        grid_spec=pltpu.PrefetchScalarGridSpec(
            num_scalar_prefetch=2,
            grid=(TILES,),
            in_specs=[
                pl.BlockSpec((TM, D), lambda i, te, nt: (i, 0)),
                pl.BlockSpec((TM, 128), lambda i, te, nt: (i, 0)),
                pl.BlockSpec((1, FF, D), lambda i, te, nt: (te[i], 0, 0)),
                pl.BlockSpec((1, 1, FF), lambda i, te, nt: (te[i], 0, 0)),
                pl.BlockSpec((1, FF, D), lambda i, te, nt: (te[i], 0, 0)),
                pl.BlockSpec((1, 1, FF), lambda i, te, nt: (te[i], 0, 0)),
                pl.BlockSpec((1, D, FF), lambda i, te, nt: (te[i], 0, 0)),
                pl.BlockSpec((1, 1, D), lambda i, te, nt: (te[i], 0, 0)),
            ],
            out_specs=pl.BlockSpec((TM, D), lambda i, te, nt: (i, 0)),
        ),
        out_shape=jax.ShapeDtypeStruct((NPAD, D), jnp.float32),
        compiler_params=pltpu.CompilerParams(
            vmem_limit_bytes=100 * 1024 * 1024),
    )(te.reshape(TE_LANES), nt.reshape(1), xs, ws,
      w1_W, w1_b.reshape(E, 1, FF), w2_W, w2_b.reshape(E, 1, FF),
      w3_W, w3_b.reshape(E, 1, D))

    combine = functools.partial(
        pl.kernel,
        out_type=(jax.ShapeDtypeStruct((N, D), jnp.float32),
                  jax.ShapeDtypeStruct((N, D), jnp.float32)),
        mesh=mesh,
        scratch_types=[
            pltpu.VMEM((NCH, CH), jnp.int32),
            pltpu.VMEM((NCH, CH), jnp.int32),
            pltpu.VMEM((CH, D), jnp.float32),
            pltpu.VMEM((CH, D), jnp.float32),
            pltpu.VMEM((CH, D), jnp.float32),
            pltpu.VMEM((CH, D), jnp.float32),
            pltpu.SemaphoreType.DMA,
            pltpu.SemaphoreType.DMA,
        ],
        compiler_params=pltpu.CompilerParams(needs_layout_passes=False),
    )(_combine_body)
    g0, g1 = combine(ys, p0w, p1w)

    TA = 512
    out = pl.pallas_call(
        _add_body,
        grid=(N // TA,),
        in_specs=[
            pl.BlockSpec((TA, D), lambda i: (i, 0)),
            pl.BlockSpec((TA, D), lambda i: (i, 0)),
        ],
        out_specs=pl.BlockSpec((TA, D), lambda i: (i, 0)),
        out_shape=jax.ShapeDtypeStruct((N, D), jnp.float32),
    )(g0, g1)

    final = out.reshape(B, S, D)
    aux_loss = jnp.asarray(0.0, dtype=jnp.float32)
    return (final, aux_loss)


# speedup vs baseline: 1.1015x; 1.0157x over previous
import functools

import jax
import jax.numpy as jnp
from jax import lax
from jax.experimental import pallas as pl
from jax.experimental.pallas import tpu as pltpu
from jax.experimental.pallas import tpu_sc as plsc

B, S, D, FF, E, K = 2, 2048, 1024, 2048, 8, 2
N = B * S
TM = 256
NPAD = N * K + E * TM
TILES = NPAD // TM
TE_LANES = 48

NC, NS = 2, 16
NW = NC * NS
TPW = N // NW
CH = 16
NCH = TPW // CH
CH2 = 32
NCH2 = TPW // CH2

_SQRT_HALF = 0.7071067811865476


_RB = 4
_PB = 8
RTN = N // _RB
PTB = N // _PB


def _router_body(x_ref, gw_ref, gb_ref, w1o_ref, w2o_ref, te_ref, nt_ref,
                 p0_ref, p1_ref, i1_s, i2_s, cnt_ref, starts_ref, carry_ref):
    i = pl.program_id(0)

    @pl.when(i == 0)
    def _():
        cnt_ref[...] = jnp.zeros_like(cnt_ref)
        carry_ref[...] = jnp.zeros_like(carry_ref)

    @pl.when(i < _RB)
    def _():
        x = x_ref[...]
        gw = gw_ref[...]
        logits = jax.lax.dot_general(
            x, gw, (((1,), (1,)), ((), ())),
            preferred_element_type=jnp.float32)
        logits = logits + gb_ref[...]
        iota = lax.broadcasted_iota(jnp.int32, logits.shape, 1)
        v1 = jnp.max(logits, axis=1, keepdims=True)
        i1 = jnp.min(jnp.where(logits == v1, iota, E), axis=1, keepdims=True)
        masked = jnp.where(iota == i1, -jnp.inf, logits)
        v2 = jnp.max(masked, axis=1, keepdims=True)
        i2 = jnp.min(jnp.where(masked == v2, iota, E), axis=1, keepdims=True)
        e1 = 1.0 / (1.0 + jnp.exp(v2 - v1))

        i1_s[pl.ds(i * RTN, RTN), :] = i1
        i2_s[pl.ds(i * RTN, RTN), :] = i2
        w1o_ref[...] = e1
        w2o_ref[...] = 1.0 - e1

        oh = ((iota == i1).astype(jnp.float32)
              + (iota == i2).astype(jnp.float32))
        cnt_ref[...] += jnp.sum(oh, axis=0, keepdims=True)

    @pl.when(i == _RB - 1)
    def _():
        cnt = cnt_ref[...]
        pc = jnp.ceil(cnt * (1.0 / TM)) * TM
        r8 = lax.broadcasted_iota(jnp.int32, (E, E), 0)
        c8 = lax.broadcasted_iota(jnp.int32, (E, E), 1)
        upper = (r8 < c8).astype(jnp.float32)
        starts = jax.lax.dot_general(
            pc, upper, (((1,), (0,)), ((), ())),
            preferred_element_type=jnp.float32)
        starts_ref[...] = starts
        eye = (r8 == c8).astype(jnp.float32)
        st_col = jax.lax.dot_general(
            eye, starts * (1.0 / TM), (((1,), (1,)), ((), ())),
            preferred_element_type=jnp.float32)
        lane = lax.broadcasted_iota(jnp.int32, (E, TE_LANES), 1)
        ge = (lane >= st_col.astype(jnp.int32)).astype(jnp.int32)
        te_ref[...] = jnp.sum(ge, axis=0, keepdims=True) - 1
        nt_ref[...] = (jnp.sum(pc, axis=1, keepdims=True)
                       * (1.0 / TM)).astype(jnp.int32)

    @pl.when(i >= _RB)
    def _():
        b = i - _RB
        iota = lax.broadcasted_iota(jnp.int32, (PTB, E), 1)
        i1 = i1_s[pl.ds(b * PTB, PTB), :]
        i2 = i2_s[pl.ds(b * PTB, PTB), :]
        oh1 = (i1 == iota).astype(jnp.float32)
        oh2 = (i2 == iota).astype(jnp.float32)
        oht = oh1 + oh2
        r_io = lax.broadcasted_iota(jnp.int32, (PTB, PTB), 0)
        c_io = lax.broadcasted_iota(jnp.int32, (PTB, PTB), 1)
        ltri = (r_io >= c_io).astype(jnp.float32)
        incl = jax.lax.dot_general(ltri, oht, (((1,), (0,)), ((), ())),
                                   preferred_element_type=jnp.float32)
        excl = incl - oht
        base = starts_ref[...] + carry_ref[...]
        p0 = jnp.sum(oh1 * (base + excl), axis=1, keepdims=True)
        p1 = jnp.sum(oh2 * (base + excl + oh1), axis=1, keepdims=True)
        p0_ref[...] = p0.astype(jnp.int32)
        p1_ref[...] = p1.astype(jnp.int32)
        carry_ref[...] += jnp.sum(oht, axis=0, keepdims=True)


def _dispatch_body(x_hbm, p0_hbm, p1_hbm, ww0_hbm, ww1_hbm, xs_hbm, ws_hbm,
                   idx0_v, idx1_v, rows0_v, rows1_v, wv0_v, wv1_v,
                   wbuf0_v, wbuf1_v, load_sem, scat_sem):
    wid = lax.axis_index("s") * NC + lax.axis_index("c")
    pltpu.sync_copy(p0_hbm.at[wid], idx0_v)
    pltpu.sync_copy(p1_hbm.at[wid], idx1_v)
    pltpu.sync_copy(ww0_hbm.at[pl.ds(wid * TPW, TPW)], wv0_v)
    pltpu.sync_copy(ww1_hbm.at[pl.ds(wid * TPW, TPW)], wv1_v)
    rows = [rows0_v, rows1_v]
    wbufs = [wbuf0_v, wbuf1_v]
    lanes = lax.iota(jnp.int32, 16)
    zeros = jnp.zeros((16,), jnp.int32)
    loads = [None] * NCH2
    scats = [None] * NCH2
    loads[0] = pltpu.async_copy(
        x_hbm.at[pl.ds(wid * TPW, CH2)], rows[0], load_sem)
    for c in range(NCH2):
        if c >= 1:
            for h in scats[c - 1]:
                h.wait()
        if c + 1 < NCH2:
            tok1 = wid * TPW + (c + 1) * CH2
            loads[c + 1] = pltpu.async_copy(
                x_hbm.at[pl.ds(tok1, CH2)], rows[(c + 1) % 2], load_sem)
        loads[c].wait()
        wb = wbufs[c % 2]
        for h in range(CH2 // 16):
            off = c * CH2 + h * 16
            plsc.store_scatter(wb, [lanes + h * 16, zeros],
                               wv0_v[pl.ds(off, 16)])
            plsc.store_scatter(wb, [lanes + CH2 + h * 16, zeros],
                               wv1_v[pl.ds(off, 16)])
        scats[c] = [
            pltpu.async_copy(rows[c % 2], xs_hbm.at[idx0_v.at[c]], scat_sem),
            pltpu.async_copy(rows[c % 2], xs_hbm.at[idx1_v.at[c]], scat_sem),
            pltpu.async_copy(wb.at[pl.ds(0, CH2)], ws_hbm.at[idx0_v.at[c]],
                             scat_sem),
            pltpu.async_copy(wb.at[pl.ds(CH2, CH2)], ws_hbm.at[idx1_v.at[c]],
                             scat_sem),
        ]
    for h in scats[NCH2 - 1]:
        h.wait()


def _ffn_body(te_ref, nt_ref, xs_ref, ws_ref, w1_ref, b1_ref, w2_ref,
              b2_ref, w3_ref, b3_ref, out_ref):
    i = pl.program_id(0)

    @pl.when(i < nt_ref[0])
    def _():
        xb = xs_ref[...]
        w1 = w1_ref[0]
        w2 = w2_ref[0]
        w3 = w3_ref[0]
        h1 = jax.lax.dot_general(xb, w1, (((1,), (1,)), ((), ())),
                                 preferred_element_type=jnp.float32)
        h1 = h1 + b1_ref[0]
        h2 = jax.lax.dot_general(xb, w2, (((1,), (1,)), ((), ())),
                                 preferred_element_type=jnp.float32)
        h2 = h2 + b2_ref[0]
        g = 0.5 * h1 * (1.0 + jax.lax.erf(h1 * _SQRT_HALF))
        h = g * h2
        part = jax.lax.dot_general(h, w3, (((1,), (1,)), ((), ())),
                                   preferred_element_type=jnp.float32)
        out_ref[...] = ws_ref[:, 0:1] * (part + b3_ref[0])


def _combine_body(ys_hbm, p0_hbm, p1_hbm, g0_hbm, g1_hbm,
                  idx0_v, idx1_v, r0a_v, r0b_v, r1a_v, r1b_v,
                  gat_sem, st_sem):
    wid = lax.axis_index("s") * NC + lax.axis_index("c")
    pltpu.sync_copy(p0_hbm.at[wid], idx0_v)
    pltpu.sync_copy(p1_hbm.at[wid], idx1_v)
    r0 = [r0a_v, r0b_v]
    r1 = [r1a_v, r1b_v]
    gats = [None] * NCH
    stores = [None] * NCH
    gats[0] = [
        pltpu.async_copy(ys_hbm.at[idx0_v.at[0]], r0[0], gat_sem),
        pltpu.async_copy(ys_hbm.at[idx1_v.at[0]], r1[0], gat_sem),
    ]
    for c in range(NCH):
        if c >= 1:
            for h in stores[c - 1]:
                h.wait()
        if c + 1 < NCH:
            gats[c + 1] = [
                pltpu.async_copy(ys_hbm.at[idx0_v.at[c + 1]],
                                 r0[(c + 1) % 2], gat_sem),
                pltpu.async_copy(ys_hbm.at[idx1_v.at[c + 1]],
                                 r1[(c + 1) % 2], gat_sem),
            ]
        for h in gats[c]:
            h.wait()
        tok0 = wid * TPW + c * CH
        stores[c] = [
            pltpu.async_copy(r0[c % 2], g0_hbm.at[pl.ds(tok0, CH)], st_sem),
            pltpu.async_copy(r1[c % 2], g1_hbm.at[pl.ds(tok0, CH)], st_sem),
        ]
    for h in stores[NCH - 1]:
        h.wait()


def _add_body(a_ref, b_ref, o_ref):
    o_ref[...] = a_ref[...] + b_ref[...]


def kernel(x, gate_W, gate_b, w1_W, w1_b, w2_W, w2_b, w3_W, w3_b):
    xf = x.reshape(N, D)

    ww1, ww2, te, nt, p0, p1 = pl.pallas_call(
        _router_body,
        grid=(_RB + _PB,),
        in_specs=[
            pl.BlockSpec((RTN, D), lambda i: (jnp.minimum(i, _RB - 1), 0)),
            pl.BlockSpec((E, D), lambda i: (0, 0)),
            pl.BlockSpec((1, E), lambda i: (0, 0)),
        ],
        out_specs=[
            pl.BlockSpec((RTN, 1), lambda i: (jnp.minimum(i, _RB - 1), 0)),
            pl.BlockSpec((RTN, 1), lambda i: (jnp.minimum(i, _RB - 1), 0)),
            pl.BlockSpec((1, TE_LANES), lambda i: (0, 0)),
            pl.BlockSpec((1, 1), lambda i: (0, 0)),
            pl.BlockSpec((PTB, 1), lambda i: (jnp.maximum(i - _RB, 0), 0)),
            pl.BlockSpec((PTB, 1), lambda i: (jnp.maximum(i - _RB, 0), 0)),
        ],
        out_shape=[
            jax.ShapeDtypeStruct((N, 1), jnp.float32),
            jax.ShapeDtypeStruct((N, 1), jnp.float32),
            jax.ShapeDtypeStruct((1, TE_LANES), jnp.int32),
            jax.ShapeDtypeStruct((1, 1), jnp.int32),
            jax.ShapeDtypeStruct((N, 1), jnp.int32),
            jax.ShapeDtypeStruct((N, 1), jnp.int32),
        ],
        scratch_shapes=[
            pltpu.VMEM((N, 1), jnp.int32),
            pltpu.VMEM((N, 1), jnp.int32),
            pltpu.VMEM((1, E), jnp.float32),
            pltpu.VMEM((1, E), jnp.float32),
            pltpu.VMEM((1, E), jnp.float32),
        ],
    )(xf, gate_W, gate_b.reshape(1, E))

    p0w = p0.reshape(NW, NCH, CH)
    p1w = p1.reshape(NW, NCH, CH)
    p0d = p0.reshape(NW, NCH2, CH2)
    p1d = p1.reshape(NW, NCH2, CH2)

    mesh = plsc.VectorSubcoreMesh(core_axis_name="c", subcore_axis_name="s")
    dispatch = functools.partial(
        pl.kernel,
        out_type=(jax.ShapeDtypeStruct((NPAD, D), jnp.float32),
                  jax.ShapeDtypeStruct((NPAD, 128), jnp.float32)),
        mesh=mesh,
        scratch_types=[
            pltpu.VMEM((NCH2, CH2), jnp.int32),
            pltpu.VMEM((NCH2, CH2), jnp.int32),
            pltpu.VMEM((CH2, D), jnp.float32),
            pltpu.VMEM((CH2, D), jnp.float32),
            pltpu.VMEM((TPW,), jnp.float32),
            pltpu.VMEM((TPW,), jnp.float32),
            pltpu.VMEM((2 * CH2, 128), jnp.float32),
            pltpu.VMEM((2 * CH2, 128), jnp.float32),
            pltpu.SemaphoreType.DMA,
            pltpu.SemaphoreType.DMA,
        ],
        compiler_params=pltpu.CompilerParams(needs_layout_passes=False),
    )(_dispatch_body)
    xs, ws = dispatch(xf, p0d, p1d, ww1.reshape(N), ww2.reshape(N))

    ys = pl.pallas_call(
        _ffn_body,
        grid_spec=pltpu.PrefetchScalarGridSpec(
            num_scalar_prefetch=2,
            grid=(TILES,),
            in_specs=[
                pl.BlockSpec((TM, D), lambda i, te, nt: (i, 0)),
                pl.BlockSpec((TM, 128), lambda i, te, nt: (i, 0)),
                pl.BlockSpec((1, FF, D), lambda i, te, nt: (te[i], 0, 0)),
                pl.BlockSpec((1, 1, FF), lambda i, te, nt: (te[i], 0, 0)),
                pl.BlockSpec((1, FF, D), lambda i, te, nt: (te[i], 0, 0)),
                pl.BlockSpec((1, 1, FF), lambda i, te, nt: (te[i], 0, 0)),
                pl.BlockSpec((1, D, FF), lambda i, te, nt: (te[i], 0, 0)),
                pl.BlockSpec((1, 1, D), lambda i, te, nt: (te[i], 0, 0)),
            ],
            out_specs=pl.BlockSpec((TM, D), lambda i, te, nt: (i, 0)),
        ),
        out_shape=jax.ShapeDtypeStruct((NPAD, D), jnp.float32),
        compiler_params=pltpu.CompilerParams(
            vmem_limit_bytes=100 * 1024 * 1024),
    )(te.reshape(TE_LANES), nt.reshape(1), xs, ws,
      w1_W, w1_b.reshape(E, 1, FF), w2_W, w2_b.reshape(E, 1, FF),
      w3_W, w3_b.reshape(E, 1, D))

    combine = functools.partial(
        pl.kernel,
        out_type=(jax.ShapeDtypeStruct((N, D), jnp.float32),
                  jax.ShapeDtypeStruct((N, D), jnp.float32)),
        mesh=mesh,
        scratch_types=[
            pltpu.VMEM((NCH, CH), jnp.int32),
            pltpu.VMEM((NCH, CH), jnp.int32),
            pltpu.VMEM((CH, D), jnp.float32),
            pltpu.VMEM((CH, D), jnp.float32),
            pltpu.VMEM((CH, D), jnp.float32),
            pltpu.VMEM((CH, D), jnp.float32),
            pltpu.SemaphoreType.DMA,
            pltpu.SemaphoreType.DMA,
        ],
        compiler_params=pltpu.CompilerParams(needs_layout_passes=False),
    )(_combine_body)
    g0, g1 = combine(ys, p0w, p1w)

    TA = 512
    out = pl.pallas_call(
        _add_body,
        grid=(N // TA,),
        in_specs=[
            pl.BlockSpec((TA, D), lambda i: (i, 0)),
            pl.BlockSpec((TA, D), lambda i: (i, 0)),
        ],
        out_specs=pl.BlockSpec((TA, D), lambda i: (i, 0)),
        out_shape=jax.ShapeDtypeStruct((N, D), jnp.float32),
    )(g0, g1)

    final = out.reshape(B, S, D)
    aux_loss = jnp.asarray(0.0, dtype=jnp.float32)
    return (final, aux_loss)


# speedup vs baseline: 1.1022x; 1.0006x over previous
import functools

import jax
import jax.numpy as jnp
from jax import lax
from jax.experimental import pallas as pl
from jax.experimental.pallas import tpu as pltpu
from jax.experimental.pallas import tpu_sc as plsc

B, S, D, FF, E, K = 2, 2048, 1024, 2048, 8, 2
N = B * S
TM = 256
NPAD = N * K + E * TM
TILES = NPAD // TM
TE_LANES = 48

NC, NS = 2, 16
NW = NC * NS
TPW = N // NW
CH = 16
NCH = TPW // CH
CH2 = 32
NCH2 = TPW // CH2

_SQRT_HALF = 0.7071067811865476


_RB = 4
_PB = 8
RTN = N // _RB
PTB = N // _PB


def _router_body(x_ref, gw_ref, gb_ref, w1o_ref, w2o_ref, te_ref, nt_ref,
                 p0_ref, p1_ref, i1_s, i2_s, cnt_ref, starts_ref, carry_ref):
    i = pl.program_id(0)

    @pl.when(i == 0)
    def _():
        cnt_ref[...] = jnp.zeros_like(cnt_ref)
        carry_ref[...] = jnp.zeros_like(carry_ref)

    @pl.when(i < _RB)
    def _():
        x = x_ref[...]
        gw = gw_ref[...]
        logits = jax.lax.dot_general(
            x, gw, (((1,), (1,)), ((), ())),
            preferred_element_type=jnp.float32)
        logits = logits + gb_ref[...]
        iota = lax.broadcasted_iota(jnp.int32, logits.shape, 1)
        v1 = jnp.max(logits, axis=1, keepdims=True)
        i1 = jnp.min(jnp.where(logits == v1, iota, E), axis=1, keepdims=True)
        masked = jnp.where(iota == i1, -jnp.inf, logits)
        v2 = jnp.max(masked, axis=1, keepdims=True)
        i2 = jnp.min(jnp.where(masked == v2, iota, E), axis=1, keepdims=True)
        e1 = 1.0 / (1.0 + jnp.exp(v2 - v1))

        i1_s[pl.ds(i * RTN, RTN), :] = i1
        i2_s[pl.ds(i * RTN, RTN), :] = i2
        w1o_ref[...] = e1
        w2o_ref[...] = 1.0 - e1

        oh = ((iota == i1).astype(jnp.float32)
              + (iota == i2).astype(jnp.float32))
        cnt_ref[...] += jnp.sum(oh, axis=0, keepdims=True)

    @pl.when(i == _RB - 1)
    def _():
        cnt = cnt_ref[...]
        pc = jnp.ceil(cnt * (1.0 / TM)) * TM
        r8 = lax.broadcasted_iota(jnp.int32, (E, E), 0)
        c8 = lax.broadcasted_iota(jnp.int32, (E, E), 1)
        upper = (r8 < c8).astype(jnp.float32)
        starts = jax.lax.dot_general(
            pc, upper, (((1,), (0,)), ((), ())),
            preferred_element_type=jnp.float32)
        starts_ref[...] = starts
        eye = (r8 == c8).astype(jnp.float32)
        st_col = jax.lax.dot_general(
            eye, starts * (1.0 / TM), (((1,), (1,)), ((), ())),
            preferred_element_type=jnp.float32)
        lane = lax.broadcasted_iota(jnp.int32, (E, TE_LANES), 1)
        ge = (lane >= st_col.astype(jnp.int32)).astype(jnp.int32)
        te_ref[...] = jnp.sum(ge, axis=0, keepdims=True) - 1
        nt_ref[...] = (jnp.sum(pc, axis=1, keepdims=True)
                       * (1.0 / TM)).astype(jnp.int32)

    @pl.when(i >= _RB)
    def _():
        b = i - _RB
        iota = lax.broadcasted_iota(jnp.int32, (PTB, E), 1)
        i1 = i1_s[pl.ds(b * PTB, PTB), :]
        i2 = i2_s[pl.ds(b * PTB, PTB), :]
        oh1 = (i1 == iota).astype(jnp.float32)
        oh2 = (i2 == iota).astype(jnp.float32)
        oht = oh1 + oh2
        r_io = lax.broadcasted_iota(jnp.int32, (PTB, PTB), 0)
        c_io = lax.broadcasted_iota(jnp.int32, (PTB, PTB), 1)
        ltri = (r_io >= c_io).astype(jnp.float32)
        incl = jax.lax.dot_general(ltri, oht, (((1,), (0,)), ((), ())),
                                   preferred_element_type=jnp.float32)
        excl = incl - oht
        base = starts_ref[...] + carry_ref[...]
        p0 = jnp.sum(oh1 * (base + excl), axis=1, keepdims=True)
        p1 = jnp.sum(oh2 * (base + excl + oh1), axis=1, keepdims=True)
        p0_ref[...] = p0.astype(jnp.int32)
        p1_ref[...] = p1.astype(jnp.int32)
        carry_ref[...] += jnp.sum(oht, axis=0, keepdims=True)


def _dispatch_body(x_hbm, p0_hbm, p1_hbm, ww0_hbm, ww1_hbm, xs_hbm, ws_hbm,
                   idx0_v, idx1_v, rows0_v, rows1_v, wv0_v, wv1_v,
                   wbuf0_v, wbuf1_v, load_sem, scat_sem):
    wid = lax.axis_index("s") * NC + lax.axis_index("c")
    pltpu.sync_copy(p0_hbm.at[wid], idx0_v)
    pltpu.sync_copy(p1_hbm.at[wid], idx1_v)
    pltpu.sync_copy(ww0_hbm.at[pl.ds(wid * TPW, TPW)], wv0_v)
    pltpu.sync_copy(ww1_hbm.at[pl.ds(wid * TPW, TPW)], wv1_v)
    rows = [rows0_v, rows1_v]
    wbufs = [wbuf0_v, wbuf1_v]
    lanes = lax.iota(jnp.int32, 16)
    zeros = jnp.zeros((16,), jnp.int32)
    loads = [None] * NCH2
    scats = [None] * NCH2
    loads[0] = pltpu.async_copy(
        x_hbm.at[pl.ds(wid * TPW, CH2)], rows[0], load_sem)
    for c in range(NCH2):
        if c >= 1:
            for h in scats[c - 1]:
                h.wait()
        if c + 1 < NCH2:
            tok1 = wid * TPW + (c + 1) * CH2
            loads[c + 1] = pltpu.async_copy(
                x_hbm.at[pl.ds(tok1, CH2)], rows[(c + 1) % 2], load_sem)
        loads[c].wait()
        wb = wbufs[c % 2]
        for h in range(CH2 // 16):
            off = c * CH2 + h * 16
            plsc.store_scatter(wb, [lanes + h * 16, zeros],
                               wv0_v[pl.ds(off, 16)])
            plsc.store_scatter(wb, [lanes + CH2 + h * 16, zeros],
                               wv1_v[pl.ds(off, 16)])
        scats[c] = [
            pltpu.async_copy(rows[c % 2], xs_hbm.at[idx0_v.at[c]], scat_sem),
            pltpu.async_copy(rows[c % 2], xs_hbm.at[idx1_v.at[c]], scat_sem),
            pltpu.async_copy(wb.at[pl.ds(0, CH2)], ws_hbm.at[idx0_v.at[c]],
                             scat_sem),
            pltpu.async_copy(wb.at[pl.ds(CH2, CH2)], ws_hbm.at[idx1_v.at[c]],
                             scat_sem),
        ]
    for h in scats[NCH2 - 1]:
        h.wait()


def _ffn_body(te_ref, nt_ref, xs_ref, ws_ref, w1_ref, b1_ref, w2_ref,
              b2_ref, w3_ref, b3_ref, out_ref):
    i = pl.program_id(0)

    @pl.when(i < nt_ref[0])
    def _():
        xb = xs_ref[...]
        w1 = w1_ref[0]
        w2 = w2_ref[0]
        w3 = w3_ref[0]
        h1 = jax.lax.dot_general(xb, w1, (((1,), (1,)), ((), ())),
                                 preferred_element_type=jnp.float32)
        h1 = h1 + b1_ref[0]
        h2 = jax.lax.dot_general(xb, w2, (((1,), (1,)), ((), ())),
                                 preferred_element_type=jnp.float32)
        h2 = h2 + b2_ref[0]
        g = 0.5 * h1 * (1.0 + jax.lax.erf(h1 * _SQRT_HALF))
        h = g * h2
        part = jax.lax.dot_general(h, w3, (((1,), (1,)), ((), ())),
                                   preferred_element_type=jnp.float32)
        out_ref[...] = ws_ref[:, 0:1] * (part + b3_ref[0])


def _combine_body(ys_hbm, p0_hbm, p1_hbm, fin_hbm,
                  idx0_v, idx1_v, r0a_v, r0b_v, r1a_v, r1b_v,
                  gat_sem, st_sem):
    wid = lax.axis_index("s") * NC + lax.axis_index("c")
    pltpu.sync_copy(p0_hbm.at[wid], idx0_v)
    pltpu.sync_copy(p1_hbm.at[wid], idx1_v)
    r0 = [r0a_v, r0b_v]
    r1 = [r1a_v, r1b_v]
    gats = [None] * NCH
    stores = [None] * NCH
    gats[0] = [
        pltpu.async_copy(ys_hbm.at[idx0_v.at[0]], r0[0], gat_sem),
        pltpu.async_copy(ys_hbm.at[idx1_v.at[0]], r1[0], gat_sem),
    ]
    for c in range(NCH):
        if c >= 1:
            for h in stores[c - 1]:
                h.wait()
        if c + 1 < NCH:
            gats[c + 1] = [
                pltpu.async_copy(ys_hbm.at[idx0_v.at[c + 1]],
                                 r0[(c + 1) % 2], gat_sem),
                pltpu.async_copy(ys_hbm.at[idx1_v.at[c + 1]],
                                 r1[(c + 1) % 2], gat_sem),
            ]
        for h in gats[c]:
            h.wait()
        a, b = r0[c % 2], r1[c % 2]
        for r in range(CH):
            def _addrow(v, _, _a=a, _b=b, _r=r):
                base = v * 64
                for k in range(4):
                    sl = pl.ds(base + k * 16, 16)
                    _a[_r, sl] = _a[_r, sl] + _b[_r, sl]
                return 0
            lax.fori_loop(0, D // 64, _addrow, 0)
        tok0 = wid * TPW + c * CH
        stores[c] = [
            pltpu.async_copy(a, fin_hbm.at[pl.ds(tok0, CH)], st_sem),
        ]
    for h in stores[NCH - 1]:
        h.wait()


def kernel(x, gate_W, gate_b, w1_W, w1_b, w2_W, w2_b, w3_W, w3_b):
    xf = x.reshape(N, D)

    ww1, ww2, te, nt, p0, p1 = pl.pallas_call(
        _router_body,
        grid=(_RB + _PB,),
        in_specs=[
            pl.BlockSpec((RTN, D), lambda i: (jnp.minimum(i, _RB - 1), 0)),
            pl.BlockSpec((E, D), lambda i: (0, 0)),
            pl.BlockSpec((1, E), lambda i: (0, 0)),
        ],
        out_specs=[
            pl.BlockSpec((RTN, 1), lambda i: (jnp.minimum(i, _RB - 1), 0)),
            pl.BlockSpec((RTN, 1), lambda i: (jnp.minimum(i, _RB - 1), 0)),
            pl.BlockSpec((1, TE_LANES), lambda i: (0, 0)),
            pl.BlockSpec((1, 1), lambda i: (0, 0)),
            pl.BlockSpec((PTB, 1), lambda i: (jnp.maximum(i - _RB, 0), 0)),
            pl.BlockSpec((PTB, 1), lambda i: (jnp.maximum(i - _RB, 0), 0)),
        ],
        out_shape=[
            jax.ShapeDtypeStruct((N, 1), jnp.float32),
            jax.ShapeDtypeStruct((N, 1), jnp.float32),
            jax.ShapeDtypeStruct((1, TE_LANES), jnp.int32),
            jax.ShapeDtypeStruct((1, 1), jnp.int32),
            jax.ShapeDtypeStruct((N, 1), jnp.int32),
            jax.ShapeDtypeStruct((N, 1), jnp.int32),
        ],
        scratch_shapes=[
            pltpu.VMEM((N, 1), jnp.int32),
            pltpu.VMEM((N, 1), jnp.int32),
            pltpu.VMEM((1, E), jnp.float32),
            pltpu.VMEM((1, E), jnp.float32),
            pltpu.VMEM((1, E), jnp.float32),
        ],
    )(xf, gate_W, gate_b.reshape(1, E))

    p0w = p0.reshape(NW, NCH, CH)
    p1w = p1.reshape(NW, NCH, CH)
    p0d = p0.reshape(NW, NCH2, CH2)
    p1d = p1.reshape(NW, NCH2, CH2)

    mesh = plsc.VectorSubcoreMesh(core_axis_name="c", subcore_axis_name="s")
    dispatch = functools.partial(
        pl.kernel,
        out_type=(jax.ShapeDtypeStruct((NPAD, D), jnp.float32),
                  jax.ShapeDtypeStruct((NPAD, 128), jnp.float32)),
        mesh=mesh,
        scratch_types=[
            pltpu.VMEM((NCH2, CH2), jnp.int32),
            pltpu.VMEM((NCH2, CH2), jnp.int32),
            pltpu.VMEM((CH2, D), jnp.float32),
            pltpu.VMEM((CH2, D), jnp.float32),
            pltpu.VMEM((TPW,), jnp.float32),
            pltpu.VMEM((TPW,), jnp.float32),
            pltpu.VMEM((2 * CH2, 128), jnp.float32),
            pltpu.VMEM((2 * CH2, 128), jnp.float32),
            pltpu.SemaphoreType.DMA,
            pltpu.SemaphoreType.DMA,
        ],
        compiler_params=pltpu.CompilerParams(needs_layout_passes=False),
    )(_dispatch_body)
    xs, ws = dispatch(xf, p0d, p1d, ww1.reshape(N), ww2.reshape(N))

    ys = pl.pallas_call(
        _ffn_body,
        grid_spec=pltpu.PrefetchScalarGridSpec(
            num_scalar_prefetch=2,
            grid=(TILES,),
            in_specs=[
                pl.BlockSpec((TM, D), lambda i, te, nt: (i, 0)),
                pl.BlockSpec((TM, 128), lambda i, te, nt: (i, 0)),
                pl.BlockSpec((1, FF, D), lambda i, te, nt: (te[i], 0, 0)),
                pl.BlockSpec((1, 1, FF), lambda i, te, nt: (te[i], 0, 0)),
                pl.BlockSpec((1, FF, D), lambda i, te, nt: (te[i], 0, 0)),
                pl.BlockSpec((1, 1, FF), lambda i, te, nt: (te[i], 0, 0)),
                pl.BlockSpec((1, D, FF), lambda i, te, nt: (te[i], 0, 0)),
                pl.BlockSpec((1, 1, D), lambda i, te, nt: (te[i], 0, 0)),
            ],
            out_specs=pl.BlockSpec((TM, D), lambda i, te, nt: (i, 0)),
        ),
        out_shape=jax.ShapeDtypeStruct((NPAD, D), jnp.float32),
        compiler_params=pltpu.CompilerParams(
            vmem_limit_bytes=100 * 1024 * 1024),
    )(te.reshape(TE_LANES), nt.reshape(1), xs, ws,
      w1_W, w1_b.reshape(E, 1, FF), w2_W, w2_b.reshape(E, 1, FF),
      w3_W, w3_b.reshape(E, 1, D))

    combine = functools.partial(
        pl.kernel,
        out_type=jax.ShapeDtypeStruct((N, D), jnp.float32),
        mesh=mesh,
        scratch_types=[
            pltpu.VMEM((NCH, CH), jnp.int32),
            pltpu.VMEM((NCH, CH), jnp.int32),
            pltpu.VMEM((CH, D), jnp.float32),
            pltpu.VMEM((CH, D), jnp.float32),
            pltpu.VMEM((CH, D), jnp.float32),
            pltpu.VMEM((CH, D), jnp.float32),
            pltpu.SemaphoreType.DMA,
            pltpu.SemaphoreType.DMA,
        ],
        compiler_params=pltpu.CompilerParams(needs_layout_passes=False),
    )(_combine_body)
    out = combine(ys, p0w, p1w)

    final = out.reshape(B, S, D)
    aux_loss = jnp.asarray(0.0, dtype=jnp.float32)
    return (final, aux_loss)
